# baseline (speedup 1.0000x reference)
import jax
import jax.numpy as jnp
from jax import lax
from jax.experimental import pallas as pl
from jax.experimental.pallas import tpu as pltpu
from jax.experimental.pallas import tpu_sc as plsc

B = 4096
K = 16
NC, NS = 2, 16
NW = NC * NS
BPW = B // NW
GROUPS = BPW // 16


def _sc_body(u_hbm, i_hbm, diff_hbm, disc_hbm, theta_hbm, x_hbm,
             u_v, i_v, disc_v, theta_v, diff_v, x_v, sem):
    wid = lax.axis_index("s") * NC + lax.axis_index("c")
    base = wid * BPW
    pltpu.sync_copy(u_hbm.at[pl.ds(base, BPW)], u_v)
    pltpu.sync_copy(i_hbm.at[pl.ds(base, BPW)], i_v)
    c1 = pltpu.async_copy(disc_hbm.at[i_v], disc_v, sem)
    c2 = pltpu.async_copy(theta_hbm.at[u_v], theta_v, sem)
    c3 = pltpu.async_copy(diff_hbm.at[i_v], diff_v, sem)
    c1.wait()
    c2.wait()
    c3.wait()
    lane = lax.iota(jnp.int32, 16)
    for g in range(GROUPS):
        acc = diff_v[pl.ds(g * 16, 16)]
        for j in range(16):
            r = g * 16 + j
            prod = disc_v[r, :] * theta_v[r, :]
            xr = jnp.sum(prod)
            acc = acc + jnp.where(lane == j, xr, 0.0)
        x_v[pl.ds(g * 16, 16)] = acc
    pltpu.sync_copy(x_v, x_hbm.at[pl.ds(base, BPW)])


_sc_x = pl.kernel(
    _sc_body,
    out_type=jax.ShapeDtypeStruct((B,), jnp.float32),
    mesh=plsc.VectorSubcoreMesh(
        core_axis_name="c", subcore_axis_name="s",
        num_cores=NC, num_subcores=NS),
    scratch_types=[
        pltpu.VMEM((BPW,), jnp.int32),
        pltpu.VMEM((BPW,), jnp.int32),
        pltpu.VMEM((BPW, K), jnp.float32),
        pltpu.VMEM((BPW, K), jnp.float32),
        pltpu.VMEM((BPW,), jnp.float32),
        pltpu.VMEM((BPW,), jnp.float32),
        pltpu.SemaphoreType.DMA,
    ],
    compiler_params=pltpu.CompilerParams(
        needs_layout_passes=False, use_tc_tiling_on_sc=False),
)


def _bce_body(x_ref, s_ref, o_ref):
    x = x_ref[...]
    s = s_ref[...]
    p = 1.0 / (1.0 + jnp.exp(-x))
    p = jnp.clip(p, 1e-12, 1.0 - 1e-12)
    t = s * jnp.log(p) + (1.0 - s) * jnp.log(1.0 - p)
    o_ref[0, 0] = -jnp.sum(t) / B


_bce = pl.pallas_call(
    _bce_body,
    out_shape=jax.ShapeDtypeStruct((1, 1), jnp.float32),
    in_specs=[pl.BlockSpec(memory_space=pltpu.VMEM),
              pl.BlockSpec(memory_space=pltpu.VMEM)],
    out_specs=pl.BlockSpec(memory_space=pltpu.SMEM),
)


def kernel(u, i, s, diff, disc, theta):
    u = u.astype(jnp.int32)
    i = i.astype(jnp.int32)
    x = _sc_x(u, i, diff.reshape(-1), disc, theta)
    xs = x.reshape(32, 128)
    sf = s.astype(jnp.float32).reshape(32, 128)
    return _bce(xs, sf)[0, 0]


# speedup vs baseline: 10.1444x; 10.1444x over previous
import jax
import jax.numpy as jnp
from jax import lax
from jax.experimental import pallas as pl
from jax.experimental.pallas import tpu as pltpu
from jax.experimental.pallas import tpu_sc as plsc

B = 4096
K = 16
N = 1000000
NC, NS = 2, 16
NW = NC * NS
BPW = B // NW
WAVE = 16
NWAVES = BPW // WAVE
TILE_A = (N // 128) * 128
LAST_A = TILE_A - 128


def _fetch_base(c):
    a = jnp.minimum(c & jnp.int32(-128), jnp.int32(LAST_A))
    return pl.multiple_of(a, 128)


def _sc_body(u_hbm, i_hbm, diff_hbm, disc_hbm, theta_hbm, x_hbm,
             u_v, i_v, dblk, tblk, fblk, dtail, ttail, ftail, x_v, sem):
    wid = lax.axis_index("s") * NC + lax.axis_index("c")
    base = wid * BPW
    pltpu.sync_copy(u_hbm.at[pl.ds(base, BPW)], u_v)
    pltpu.sync_copy(i_hbm.at[pl.ds(base, BPW)], i_v)
    pltpu.sync_copy(disc_hbm.at[:, pl.ds(TILE_A, 64)], dtail.at[pl.ds(0, K), :])
    pltpu.sync_copy(theta_hbm.at[:, pl.ds(TILE_A, 64)], ttail.at[pl.ds(0, K), :])
    pltpu.sync_copy(diff_hbm.at[:, pl.ds(TILE_A, 64)], ftail.at[pl.ds(0, 1), :])
    lane = lax.iota(jnp.int32, 16)

    def wave(w, _):
        iv = i_v[pl.ds(w * WAVE, WAVE)]
        uv = u_v[pl.ds(w * WAVE, WAVE)]
        copies = []
        for j in range(WAVE):
            ai = _fetch_base(iv[j])
            au = _fetch_base(uv[j])
            copies.append(pltpu.async_copy(
                disc_hbm.at[:, pl.ds(ai, 128)],
                dblk.at[j, pl.ds(0, K), :], sem))
            copies.append(pltpu.async_copy(
                theta_hbm.at[:, pl.ds(au, 128)],
                tblk.at[j, pl.ds(0, K), :], sem))
            copies.append(pltpu.async_copy(
                diff_hbm.at[:, pl.ds(ai, 128)],
                fblk.at[j, pl.ds(0, 1), :], sem))
        for c in copies:
            c.wait()
        res = jnp.zeros((16,), jnp.float32)
        for j in range(WAVE):
            ci = iv[j]
            cu = uv[j]
            tail_i = ci >= jnp.int32(TILE_A)
            tail_u = cu >= jnp.int32(TILE_A)
            offi = jnp.minimum(ci - _fetch_base(ci), jnp.int32(127))
            offu = jnp.minimum(cu - _fetch_base(cu), jnp.int32(127))
            oti = jnp.clip(ci - jnp.int32(TILE_A), 0, 63)
            otu = jnp.clip(cu - jnp.int32(TILE_A), 0, 63)
            acc = jnp.where(tail_i, ftail[0, pl.ds(oti, 16)],
                            fblk[j, 0, pl.ds(offi, 16)])
            for k in range(K):
                d = jnp.where(tail_i, dtail[k, pl.ds(oti, 16)],
                              dblk[j, k, pl.ds(offi, 16)])
                t = jnp.where(tail_u, ttail[k, pl.ds(otu, 16)],
                              tblk[j, k, pl.ds(offu, 16)])
                acc = acc + d * t
            res = jnp.where(lane == j, acc[0], res)
        x_v[pl.ds(w * WAVE, WAVE)] = res
        return ()

    lax.fori_loop(0, NWAVES, wave, ())
    pltpu.sync_copy(x_v, x_hbm.at[pl.ds(base, BPW)])


def _bce_body(x_ref, s_ref, o_ref):
    x = x_ref[...]
    s = s_ref[...]
    p = 1.0 / (1.0 + jnp.exp(-x))
    p = jnp.clip(p, 1e-12, 1.0 - 1e-12)
    t = s * jnp.log(p) + (1.0 - s) * jnp.log(1.0 - p)
    o_ref[0, 0] = -jnp.sum(t) / B


_bce = pl.pallas_call(
    _bce_body,
    out_shape=jax.ShapeDtypeStruct((1, 1), jnp.float32),
    in_specs=[pl.BlockSpec(memory_space=pltpu.VMEM),
              pl.BlockSpec(memory_space=pltpu.VMEM)],
    out_specs=pl.BlockSpec(memory_space=pltpu.SMEM),
)


def _make_sc_x():
    return pl.kernel(
        _sc_body,
        out_type=jax.ShapeDtypeStruct((B,), jnp.float32),
        mesh=plsc.VectorSubcoreMesh(
            core_axis_name="c", subcore_axis_name="s",
            num_cores=NC, num_subcores=NS),
        scratch_types=[
            pltpu.VMEM((BPW,), jnp.int32),
            pltpu.VMEM((BPW,), jnp.int32),
            pltpu.VMEM((WAVE, K + 1, 128), jnp.float32),
            pltpu.VMEM((WAVE, K + 1, 128), jnp.float32),
            pltpu.VMEM((WAVE, 2, 128), jnp.float32),
            pltpu.VMEM((K + 1, 64), jnp.float32),
            pltpu.VMEM((K + 1, 64), jnp.float32),
            pltpu.VMEM((2, 64), jnp.float32),
            pltpu.VMEM((BPW,), jnp.float32),
            pltpu.SemaphoreType.DMA,
        ],
    )


_sc_x = None


def kernel(u, i, s, diff, disc, theta):
    global _sc_x
    if _sc_x is None:
        _sc_x = _make_sc_x()
    u = u.astype(jnp.int32)
    i = i.astype(jnp.int32)
    x = _sc_x(u, i, diff.T, disc.T, theta.T)
    xs = x.reshape(32, 128)
    sf = s.astype(jnp.float32).reshape(32, 128)
    return _bce(xs, sf)[0, 0]


# speedup vs baseline: 12.2991x; 1.2124x over previous
import jax
import jax.numpy as jnp
from jax import lax
from jax.experimental import pallas as pl
from jax.experimental.pallas import tpu as pltpu
from jax.experimental.pallas import tpu_sc as plsc

B = 4096
K = 16
N = 1000000
NC, NS = 2, 16
NW = NC * NS
BPW = B // NW
SW = 8
NP = BPW // (2 * SW)
TILE_A = (N // 128) * 128
LAST_A = TILE_A - 128


def _fetch_base(c):
    a = jnp.minimum(c & jnp.int32(-128), jnp.int32(LAST_A))
    return pl.multiple_of(a, 128)


def _sc_body(u_hbm, i_hbm, diff_hbm, disc_hbm, theta_hbm, x_hbm,
             u_v, i_v, dblk, tblk, fblk, dtail, ttail, ftail,
             dfix, tfix, ffix, x_v, sem0, sem1):
    wid = lax.axis_index("s") * NC + lax.axis_index("c")
    base = wid * BPW
    pltpu.sync_copy(u_hbm.at[pl.ds(base, BPW)], u_v)
    pltpu.sync_copy(i_hbm.at[pl.ds(base, BPW)], i_v)
    pltpu.sync_copy(disc_hbm.at[:, pl.ds(TILE_A, 64)], dtail.at[pl.ds(0, K), :])
    pltpu.sync_copy(theta_hbm.at[:, pl.ds(TILE_A, 64)], ttail.at[pl.ds(0, K), :])
    pltpu.sync_copy(diff_hbm.at[:, pl.ds(TILE_A, 64)], ftail.at[pl.ds(0, 1), :])
    lane = lax.iota(jnp.int32, 16)
    sems = (sem0, sem1)

    def fire(iv, uv, half, buf, sem):
        for j in range(SW):
            ai = _fetch_base(iv[half * SW + j])
            au = _fetch_base(uv[half * SW + j])
            pltpu.async_copy(disc_hbm.at[:, pl.ds(ai, 128)],
                             dblk.at[buf, j, pl.ds(0, K), :], sem)
            pltpu.async_copy(theta_hbm.at[:, pl.ds(au, 128)],
                             tblk.at[buf, j, pl.ds(0, K), :], sem)
            pltpu.async_copy(diff_hbm.at[:, pl.ds(ai, 128)],
                             fblk.at[buf, j, pl.ds(0, 1), :], sem)

    def drain(buf, sem):
        for j in range(SW):
            pltpu.make_async_copy(disc_hbm.at[:, pl.ds(0, 128)],
                                  dblk.at[buf, j, pl.ds(0, K), :], sem).wait()
            pltpu.make_async_copy(theta_hbm.at[:, pl.ds(0, 128)],
                                  tblk.at[buf, j, pl.ds(0, K), :], sem).wait()
            pltpu.make_async_copy(diff_hbm.at[:, pl.ds(0, 128)],
                                  fblk.at[buf, j, pl.ds(0, 1), :], sem).wait()

    def compute(iv, uv, half, buf, res):
        for j in range(SW):
            ci = iv[half * SW + j]
            cu = uv[half * SW + j]
            offi = jnp.minimum(ci - _fetch_base(ci), jnp.int32(127))
            offu = jnp.minimum(cu - _fetch_base(cu), jnp.int32(127))
            acc = fblk[buf, j, 0, pl.ds(offi, 16)]
            for k in range(K):
                acc = acc + (dblk[buf, j, k, pl.ds(offi, 16)]
                             * tblk[buf, j, k, pl.ds(offu, 16)])
            res = jnp.where(lane == half * SW + j, acc[0], res)
        return res

    iv0 = i_v[pl.ds(0, 16)]
    uv0 = u_v[pl.ds(0, 16)]
    fire(iv0, uv0, 0, 0, sem0)

    def step(p, _):
        iv = i_v[pl.ds(p * 16, 16)]
        uv = u_v[pl.ds(p * 16, 16)]
        fire(iv, uv, 1, 1, sem1)
        drain(0, sem0)
        res = compute(iv, uv, 0, 0, jnp.zeros((16,), jnp.float32))

        @pl.when(p < NP - 1)
        def _():
            nxt = i_v[pl.ds((p + 1) * 16, 16)]
            nxu = u_v[pl.ds((p + 1) * 16, 16)]
            fire(nxt, nxu, 0, 0, sem0)

        drain(1, sem1)
        res = compute(iv, uv, 1, 1, res)
        x_v[pl.ds(p * 16, 16)] = res
        return ()

    lax.fori_loop(0, NP, step, ())

    def fix(p, _):
        iv = i_v[pl.ds(p * 16, 16)]
        uv = u_v[pl.ds(p * 16, 16)]
        for j in range(16):
            ci = iv[j]
            cu = uv[j]
            tail_i = ci >= jnp.int32(TILE_A)
            tail_u = cu >= jnp.int32(TILE_A)

            @pl.when(tail_i | tail_u)
            def _(ci=ci, cu=cu, tail_i=tail_i, tail_u=tail_u, j=j):
                ai = _fetch_base(ci)
                au = _fetch_base(cu)
                pltpu.sync_copy(disc_hbm.at[:, pl.ds(ai, 128)],
                                dfix.at[pl.ds(0, K), :])
                pltpu.sync_copy(theta_hbm.at[:, pl.ds(au, 128)],
                                tfix.at[pl.ds(0, K), :])
                pltpu.sync_copy(diff_hbm.at[:, pl.ds(ai, 128)],
                                ffix.at[pl.ds(0, 1), :])
                offi = jnp.minimum(ci - ai, jnp.int32(127))
                offu = jnp.minimum(cu - au, jnp.int32(127))
                oti = jnp.clip(ci - jnp.int32(TILE_A), 0, 63)
                otu = jnp.clip(cu - jnp.int32(TILE_A), 0, 63)
                acc = jnp.where(tail_i, ftail[0, pl.ds(oti, 16)],
                                ffix[0, pl.ds(offi, 16)])
                for k in range(K):
                    d = jnp.where(tail_i, dtail[k, pl.ds(oti, 16)],
                                  dfix[k, pl.ds(offi, 16)])
                    t = jnp.where(tail_u, ttail[k, pl.ds(otu, 16)],
                                  tfix[k, pl.ds(offu, 16)])
                    acc = acc + d * t
                cur = x_v[pl.ds(p * 16, 16)]
                x_v[pl.ds(p * 16, 16)] = jnp.where(lane == j, acc[0], cur)

        return ()

    lax.fori_loop(0, NP, fix, ())
    pltpu.sync_copy(x_v, x_hbm.at[pl.ds(base, BPW)])


def _bce_body(x_ref, s_ref, o_ref):
    x = x_ref[...]
    s = s_ref[...]
    p = 1.0 / (1.0 + jnp.exp(-x))
    p = jnp.clip(p, 1e-12, 1.0 - 1e-12)
    t = s * jnp.log(p) + (1.0 - s) * jnp.log(1.0 - p)
    o_ref[0, 0] = -jnp.sum(t) / B


_bce = pl.pallas_call(
    _bce_body,
    out_shape=jax.ShapeDtypeStruct((1, 1), jnp.float32),
    in_specs=[pl.BlockSpec(memory_space=pltpu.VMEM),
              pl.BlockSpec(memory_space=pltpu.VMEM)],
    out_specs=pl.BlockSpec(memory_space=pltpu.SMEM),
)


def _make_sc_x():
    return pl.kernel(
        _sc_body,
        out_type=jax.ShapeDtypeStruct((B,), jnp.float32),
        mesh=plsc.VectorSubcoreMesh(
            core_axis_name="c", subcore_axis_name="s",
            num_cores=NC, num_subcores=NS),
        scratch_types=[
            pltpu.VMEM((BPW,), jnp.int32),
            pltpu.VMEM((BPW,), jnp.int32),
            pltpu.VMEM((2, SW, K + 1, 128), jnp.float32),
            pltpu.VMEM((2, SW, K + 1, 128), jnp.float32),
            pltpu.VMEM((2, SW, 2, 128), jnp.float32),
            pltpu.VMEM((K + 1, 64), jnp.float32),
            pltpu.VMEM((K + 1, 64), jnp.float32),
            pltpu.VMEM((2, 64), jnp.float32),
            pltpu.VMEM((K + 1, 128), jnp.float32),
            pltpu.VMEM((K + 1, 128), jnp.float32),
            pltpu.VMEM((2, 128), jnp.float32),
            pltpu.VMEM((BPW,), jnp.float32),
            pltpu.SemaphoreType.DMA,
            pltpu.SemaphoreType.DMA,
        ],
    )


_sc_x = None


def kernel(u, i, s, diff, disc, theta):
    global _sc_x
    if _sc_x is None:
        _sc_x = _make_sc_x()
    u = u.astype(jnp.int32)
    i = i.astype(jnp.int32)
    x = _sc_x(u, i, diff.T, disc.T, theta.T)
    xs = x.reshape(32, 128)
    sf = s.astype(jnp.float32).reshape(32, 128)
    return _bce(xs, sf)[0, 0]
